# baseline (speedup 1.0000x reference)
import functools

import jax
import jax.numpy as jnp
from jax import lax
from jax.experimental import pallas as pl
from jax.experimental.pallas import tpu as pltpu
from jax.experimental.pallas import tpu_sc as plsc

N = 10000
E = 320000
D = 128

NC = 2
NS = 16
NW = NC * NS

CHUNK = 128
TRIPS = 80
GRP = 16
NCHUNK = NW * TRIPS
NPAD = 10240
ROWS_PER_TILE = NPAD // NS

_PREC = jax.lax.Precision.HIGHEST


def _spmm_sc(h, src, dst, zeros):
    mesh = plsc.VectorSubcoreMesh(core_axis_name="c", subcore_axis_name="s")

    @functools.partial(
        pl.kernel,
        out_type=jax.ShapeDtypeStruct((NC * NPAD, D), jnp.float32),
        mesh=mesh,
        scratch_types=[
            pltpu.VMEM((CHUNK,), jnp.int32),
            pltpu.VMEM((CHUNK,), jnp.int32),
            pltpu.VMEM((CHUNK, D), jnp.float32),
            pltpu.VMEM_SHARED((NPAD, D), jnp.float32),
            pltpu.SemaphoreType.DMA,
        ],
    )
    def spmm(h_hbm, src_hbm, dst_hbm, zeros_hbm, out_hbm,
             src_v, dst_v, rows_v, acc_sh, sem):
        c = lax.axis_index("c")
        s = lax.axis_index("s")
        wid = s * NC + c

        pltpu.sync_copy(zeros_hbm, acc_sh.at[pl.ds(s * ROWS_PER_TILE, ROWS_PER_TILE)])
        plsc.subcore_barrier()

        def body(t, carry):
            off = (t * NW + wid) * CHUNK
            pltpu.sync_copy(src_hbm.at[pl.ds(off, CHUNK)], src_v)
            pltpu.sync_copy(dst_hbm.at[pl.ds(off, CHUNK)], dst_v)
            pltpu.async_copy(h_hbm.at[src_v], rows_v, sem).wait()
            pltpu.sync_copy(rows_v, acc_sh.at[dst_v], add=True)
            return carry

        lax.fori_loop(0, TRIPS, body, 0)
        plsc.subcore_barrier()

        pltpu.sync_copy(
            acc_sh.at[pl.ds(s * ROWS_PER_TILE, ROWS_PER_TILE)],
            out_hbm.at[pl.ds(c * NPAD + s * ROWS_PER_TILE, ROWS_PER_TILE)],
        )

    return spmm(h, src, dst, zeros)


_R = 1000


def _mm_body(x_ref, w_ref, o_ref):
    o_ref[...] = jnp.dot(x_ref[...], w_ref[...],
                         preferred_element_type=jnp.float32, precision=_PREC)


def _mm(x, w):
    return pl.pallas_call(
        _mm_body,
        grid=(N // _R,),
        in_specs=[
            pl.BlockSpec((_R, D), lambda i: (i, 0)),
            pl.BlockSpec((D, D), lambda i: (0, 0)),
        ],
        out_specs=pl.BlockSpec((_R, D), lambda i: (i, 0)),
        out_shape=jax.ShapeDtypeStruct((N, D), jnp.float32),
    )(x, w)


def _combine_mm_body(p0_ref, p1_ref, b_ref, w_ref, o_ref):
    h = jax.nn.relu(p0_ref[...] + p1_ref[...] + b_ref[...])
    o_ref[...] = jnp.dot(h, w_ref[...],
                         preferred_element_type=jnp.float32, precision=_PREC)


def _combine_mm(p0, p1, b, w):
    return pl.pallas_call(
        _combine_mm_body,
        grid=(N // _R,),
        in_specs=[
            pl.BlockSpec((_R, D), lambda i: (i, 0)),
            pl.BlockSpec((_R, D), lambda i: (i, 0)),
            pl.BlockSpec((1, D), lambda i: (0, 0)),
            pl.BlockSpec((D, D), lambda i: (0, 0)),
        ],
        out_specs=pl.BlockSpec((_R, D), lambda i: (i, 0)),
        out_shape=jax.ShapeDtypeStruct((N, D), jnp.float32),
    )(p0, p1, b, w)


def _ff_body(q0_ref, q1_ref, b2_ref, wf1_ref, bf1_ref, wf2_ref, bf2_ref,
             wf3_ref, bf3_ref, ws_ref, bs_ref, o_ref):
    h = q0_ref[...] + q1_ref[...] + b2_ref[...]
    blk = jax.nn.relu(jnp.dot(h, wf1_ref[...],
                              preferred_element_type=jnp.float32,
                              precision=_PREC) + bf1_ref[...])
    blk = jax.nn.relu(jnp.dot(blk, wf2_ref[...],
                              preferred_element_type=jnp.float32,
                              precision=_PREC) + bf2_ref[...])
    blk = jnp.dot(blk, wf3_ref[...],
                  preferred_element_type=jnp.float32, precision=_PREC) + bf3_ref[...]
    sc = jnp.dot(h, ws_ref[...],
                 preferred_element_type=jnp.float32, precision=_PREC) + bs_ref[...]
    o_ref[...] = blk + sc


def _ff(q0, q1, b2, wf1, bf1, wf2, bf2, wf3, bf3, ws, bs):
    row = pl.BlockSpec((_R, D), lambda i: (i, 0))
    mat = pl.BlockSpec((D, D), lambda i: (0, 0))
    vec = pl.BlockSpec((1, D), lambda i: (0, 0))
    return pl.pallas_call(
        _ff_body,
        grid=(N // _R,),
        in_specs=[row, row, vec, mat, vec, mat, vec, mat, vec, mat, vec],
        out_specs=row,
        out_shape=jax.ShapeDtypeStruct((N, D), jnp.float32),
    )(q0, q1, b2, wf1, bf1, wf2, bf2, wf3, bf3, ws, bs)


def kernel(x, edge_index, W1, b1, W2, b2, Wf1, bf1, Wf2, bf2, Wf3, bf3, Ws, bs):
    npad_edges = NCHUNK * CHUNK - E
    src = jnp.concatenate([edge_index[0], jnp.zeros((npad_edges,), jnp.int32)])
    dst = jnp.concatenate(
        [edge_index[1], N + (jnp.arange(npad_edges, dtype=jnp.int32) % (NPAD - N))]
    )
    zeros = jnp.zeros((ROWS_PER_TILE, D), jnp.float32)

    h1 = _mm(x, W1)
    p = _spmm_sc(h1, src, dst, zeros)
    h2 = _combine_mm(p[:N], p[NPAD:NPAD + N], b1.reshape(1, D), W2)
    q = _spmm_sc(h2, src, dst, zeros)
    out = _ff(q[:N], q[NPAD:NPAD + N], b2.reshape(1, D),
              Wf1, bf1.reshape(1, D), Wf2, bf2.reshape(1, D),
              Wf3, bf3.reshape(1, D), Ws, bs.reshape(1, D))
    return out


# speedup vs baseline: 1.8816x; 1.8816x over previous
import functools

import jax
import jax.numpy as jnp
from jax import lax
from jax.experimental import pallas as pl
from jax.experimental.pallas import tpu as pltpu
from jax.experimental.pallas import tpu_sc as plsc

N = 10000
E = 320000
D = 128

NC = 2
NS = 16
NW = NC * NS

CHUNK = 128
NCHUNK = E // CHUNK
TRIPS = -(-NCHUNK // NW)
NPAD = 10240
ROWS_PER_TILE = NPAD // NS

_PREC = jax.lax.Precision.HIGHEST


def _spmm_sc(h, src, dst, zeros):
    mesh = plsc.VectorSubcoreMesh(core_axis_name="c", subcore_axis_name="s")

    @functools.partial(
        pl.kernel,
        out_type=jax.ShapeDtypeStruct((NC * NPAD, D), jnp.float32),
        mesh=mesh,
        scratch_types=[
            pltpu.VMEM((CHUNK,), jnp.int32),
            pltpu.VMEM((CHUNK,), jnp.int32),
            pltpu.VMEM((CHUNK, D), jnp.float32),
            pltpu.VMEM_SHARED((NPAD, D), jnp.float32),
            pltpu.SemaphoreType.DMA,
        ],
    )
    def spmm(h_hbm, src_hbm, dst_hbm, zeros_hbm, out_hbm,
             src_v, dst_v, rows_v, acc_sh, sem):
        c = lax.axis_index("c")
        s = lax.axis_index("s")
        wid = s * NC + c

        pltpu.sync_copy(zeros_hbm, acc_sh.at[pl.ds(s * ROWS_PER_TILE, ROWS_PER_TILE)])
        plsc.subcore_barrier()

        def body(t, carry):
            k = t * NW + wid

            @pl.when(k < NCHUNK)
            def _():
                off = k * CHUNK
                pltpu.sync_copy(src_hbm.at[pl.ds(off, CHUNK)], src_v)
                pltpu.sync_copy(dst_hbm.at[pl.ds(off, CHUNK)], dst_v)
                pltpu.async_copy(h_hbm.at[src_v], rows_v, sem).wait()
                pltpu.sync_copy(rows_v, acc_sh.at[dst_v], add=True)

            return carry

        lax.fori_loop(0, TRIPS, body, 0)
        plsc.subcore_barrier()

        pltpu.sync_copy(
            acc_sh.at[pl.ds(s * ROWS_PER_TILE, ROWS_PER_TILE)],
            out_hbm.at[pl.ds(c * NPAD + s * ROWS_PER_TILE, ROWS_PER_TILE)],
        )

    return spmm(h, src, dst, zeros)


_R = 1000


def _mm_body(x_ref, w_ref, o_ref):
    o_ref[...] = jnp.dot(x_ref[...], w_ref[...],
                         preferred_element_type=jnp.float32, precision=_PREC)


def _mm(x, w):
    return pl.pallas_call(
        _mm_body,
        grid=(N // _R,),
        in_specs=[
            pl.BlockSpec((_R, D), lambda i: (i, 0)),
            pl.BlockSpec((D, D), lambda i: (0, 0)),
        ],
        out_specs=pl.BlockSpec((_R, D), lambda i: (i, 0)),
        out_shape=jax.ShapeDtypeStruct((N, D), jnp.float32),
    )(x, w)


def _combine_mm_body(p0_ref, p1_ref, b_ref, w_ref, o_ref):
    h = jax.nn.relu(p0_ref[...] + p1_ref[...] + b_ref[...])
    o_ref[...] = jnp.dot(h, w_ref[...],
                         preferred_element_type=jnp.float32, precision=_PREC)


def _combine_mm(p0, p1, b, w):
    return pl.pallas_call(
        _combine_mm_body,
        grid=(N // _R,),
        in_specs=[
            pl.BlockSpec((_R, D), lambda i: (i, 0)),
            pl.BlockSpec((_R, D), lambda i: (i, 0)),
            pl.BlockSpec((1, D), lambda i: (0, 0)),
            pl.BlockSpec((D, D), lambda i: (0, 0)),
        ],
        out_specs=pl.BlockSpec((_R, D), lambda i: (i, 0)),
        out_shape=jax.ShapeDtypeStruct((N, D), jnp.float32),
    )(p0, p1, b, w)


def _ff_body(q0_ref, q1_ref, b2_ref, wf1_ref, bf1_ref, wf2_ref, bf2_ref,
             wf3_ref, bf3_ref, ws_ref, bs_ref, o_ref):
    h = q0_ref[...] + q1_ref[...] + b2_ref[...]
    blk = jax.nn.relu(jnp.dot(h, wf1_ref[...],
                              preferred_element_type=jnp.float32,
                              precision=_PREC) + bf1_ref[...])
    blk = jax.nn.relu(jnp.dot(blk, wf2_ref[...],
                              preferred_element_type=jnp.float32,
                              precision=_PREC) + bf2_ref[...])
    blk = jnp.dot(blk, wf3_ref[...],
                  preferred_element_type=jnp.float32, precision=_PREC) + bf3_ref[...]
    sc = jnp.dot(h, ws_ref[...],
                 preferred_element_type=jnp.float32, precision=_PREC) + bs_ref[...]
    o_ref[...] = blk + sc


def _ff(q0, q1, b2, wf1, bf1, wf2, bf2, wf3, bf3, ws, bs):
    row = pl.BlockSpec((_R, D), lambda i: (i, 0))
    mat = pl.BlockSpec((D, D), lambda i: (0, 0))
    vec = pl.BlockSpec((1, D), lambda i: (0, 0))
    return pl.pallas_call(
        _ff_body,
        grid=(N // _R,),
        in_specs=[row, row, vec, mat, vec, mat, vec, mat, vec, mat, vec],
        out_specs=row,
        out_shape=jax.ShapeDtypeStruct((N, D), jnp.float32),
    )(q0, q1, b2, wf1, bf1, wf2, bf2, wf3, bf3, ws, bs)


def kernel(x, edge_index, W1, b1, W2, b2, Wf1, bf1, Wf2, bf2, Wf3, bf3, Ws, bs):
    src = edge_index[0]
    dst = edge_index[1]
    zeros = jnp.zeros((ROWS_PER_TILE, D), jnp.float32)

    h1 = _mm(x, W1)
    p = _spmm_sc(h1, src, dst, zeros)
    h2 = _combine_mm(p[:N], p[NPAD:NPAD + N], b1.reshape(1, D), W2)
    q = _spmm_sc(h2, src, dst, zeros)
    out = _ff(q[:N], q[NPAD:NPAD + N], b2.reshape(1, D),
              Wf1, bf1.reshape(1, D), Wf2, bf2.reshape(1, D),
              Wf3, bf3.reshape(1, D), Ws, bs.reshape(1, D))
    return out


# speedup vs baseline: 2.1233x; 1.1285x over previous
import functools

import jax
import jax.numpy as jnp
from jax import lax
from jax.experimental import pallas as pl
from jax.experimental.pallas import tpu as pltpu
from jax.experimental.pallas import tpu_sc as plsc

N = 10000
E = 320000
D = 128

NC = 2
NS = 16
NW = NC * NS

CHUNK = 128
NCHUNK = E // CHUNK
TRIPS = -(-NCHUNK // NW)
NPAD = 10240
ROWS_PER_TILE = NPAD // NS

_PREC = jax.lax.Precision.HIGHEST


def _spmm_sc(h, edges, zeros):
    mesh = plsc.VectorSubcoreMesh(core_axis_name="c", subcore_axis_name="s")

    @functools.partial(
        pl.kernel,
        out_type=jax.ShapeDtypeStruct((NC * NPAD, D), jnp.float32),
        mesh=mesh,
        scratch_types=[
            pltpu.VMEM((2, CHUNK), jnp.int32),
            pltpu.VMEM((CHUNK, D), jnp.float32),
            pltpu.VMEM_SHARED((NPAD, D), jnp.float32),
            pltpu.SemaphoreType.DMA,
        ],
    )
    def spmm(h_hbm, edge_hbm, zeros_hbm, out_hbm,
             idx_v, rows_v, acc_sh, sem):
        c = lax.axis_index("c")
        s = lax.axis_index("s")
        wid = s * NC + c

        pltpu.sync_copy(zeros_hbm, acc_sh.at[pl.ds(s * ROWS_PER_TILE, ROWS_PER_TILE)])
        plsc.subcore_barrier()

        def body(t, carry):
            k = t * NW + wid

            @pl.when(k < NCHUNK)
            def _():
                pltpu.sync_copy(edge_hbm.at[k], idx_v)
                pltpu.async_copy(h_hbm.at[idx_v.at[0]], rows_v, sem).wait()
                pltpu.sync_copy(rows_v, acc_sh.at[idx_v.at[1]], add=True)

            return carry

        lax.fori_loop(0, TRIPS, body, 0)
        plsc.subcore_barrier()

        pltpu.sync_copy(
            acc_sh.at[pl.ds(s * ROWS_PER_TILE, ROWS_PER_TILE)],
            out_hbm.at[pl.ds(c * NPAD + s * ROWS_PER_TILE, ROWS_PER_TILE)],
        )

    return spmm(h, edges, zeros)


_R = 1000


def _mm_body(x_ref, w_ref, o_ref):
    o_ref[...] = jnp.dot(x_ref[...], w_ref[...],
                         preferred_element_type=jnp.float32, precision=_PREC)


def _mm(x, w):
    return pl.pallas_call(
        _mm_body,
        grid=(N // _R,),
        in_specs=[
            pl.BlockSpec((_R, D), lambda i: (i, 0)),
            pl.BlockSpec((D, D), lambda i: (0, 0)),
        ],
        out_specs=pl.BlockSpec((_R, D), lambda i: (i, 0)),
        out_shape=jax.ShapeDtypeStruct((N, D), jnp.float32),
    )(x, w)


def _combine_mm_body(p0_ref, p1_ref, b_ref, w_ref, o_ref):
    h = jax.nn.relu(p0_ref[...] + p1_ref[...] + b_ref[...])
    o_ref[...] = jnp.dot(h, w_ref[...],
                         preferred_element_type=jnp.float32, precision=_PREC)


def _combine_mm(p0, p1, b, w):
    return pl.pallas_call(
        _combine_mm_body,
        grid=(N // _R,),
        in_specs=[
            pl.BlockSpec((_R, D), lambda i: (i, 0)),
            pl.BlockSpec((_R, D), lambda i: (i, 0)),
            pl.BlockSpec((1, D), lambda i: (0, 0)),
            pl.BlockSpec((D, D), lambda i: (0, 0)),
        ],
        out_specs=pl.BlockSpec((_R, D), lambda i: (i, 0)),
        out_shape=jax.ShapeDtypeStruct((N, D), jnp.float32),
    )(p0, p1, b, w)


def _ff_body(q0_ref, q1_ref, b2_ref, wf1_ref, bf1_ref, wf2_ref, bf2_ref,
             wf3_ref, bf3_ref, ws_ref, bs_ref, o_ref):
    h = q0_ref[...] + q1_ref[...] + b2_ref[...]
    blk = jax.nn.relu(jnp.dot(h, wf1_ref[...],
                              preferred_element_type=jnp.float32,
                              precision=_PREC) + bf1_ref[...])
    blk = jax.nn.relu(jnp.dot(blk, wf2_ref[...],
                              preferred_element_type=jnp.float32,
                              precision=_PREC) + bf2_ref[...])
    blk = jnp.dot(blk, wf3_ref[...],
                  preferred_element_type=jnp.float32, precision=_PREC) + bf3_ref[...]
    sc = jnp.dot(h, ws_ref[...],
                 preferred_element_type=jnp.float32, precision=_PREC) + bs_ref[...]
    o_ref[...] = blk + sc


def _ff(q0, q1, b2, wf1, bf1, wf2, bf2, wf3, bf3, ws, bs):
    row = pl.BlockSpec((_R, D), lambda i: (i, 0))
    mat = pl.BlockSpec((D, D), lambda i: (0, 0))
    vec = pl.BlockSpec((1, D), lambda i: (0, 0))
    return pl.pallas_call(
        _ff_body,
        grid=(N // _R,),
        in_specs=[row, row, vec, mat, vec, mat, vec, mat, vec, mat, vec],
        out_specs=row,
        out_shape=jax.ShapeDtypeStruct((N, D), jnp.float32),
    )(q0, q1, b2, wf1, bf1, wf2, bf2, wf3, bf3, ws, bs)


def kernel(x, edge_index, W1, b1, W2, b2, Wf1, bf1, Wf2, bf2, Wf3, bf3, Ws, bs):
    edges = jnp.stack(
        [edge_index[0].reshape(NCHUNK, CHUNK), edge_index[1].reshape(NCHUNK, CHUNK)],
        axis=1,
    )
    zeros = jnp.zeros((ROWS_PER_TILE, D), jnp.float32)

    h1 = _mm(x, W1)
    p = _spmm_sc(h1, edges, zeros)
    h2 = _combine_mm(p[:N], p[NPAD:NPAD + N], b1.reshape(1, D), W2)
    q = _spmm_sc(h2, edges, zeros)
    out = _ff(q[:N], q[NPAD:NPAD + N], b2.reshape(1, D),
              Wf1, bf1.reshape(1, D), Wf2, bf2.reshape(1, D),
              Wf3, bf3.reshape(1, D), Ws, bs.reshape(1, D))
    return out


# speedup vs baseline: 2.7439x; 1.2923x over previous
import functools

import jax
import jax.numpy as jnp
from jax import lax
from jax.experimental import pallas as pl
from jax.experimental.pallas import tpu as pltpu
from jax.experimental.pallas import tpu_sc as plsc

N = 10000
E = 320000
D = 128

NC = 2
NS = 16
NW = NC * NS

CHUNK = 128
NPAIR = E // (2 * CHUNK)
TRIPS = -(-NPAIR // NW)
NPAD = 10240
ROWS_PER_TILE = NPAD // NS

_PREC = jax.lax.Precision.HIGHEST


def _spmm_sc(h, edges, zeros):
    mesh = plsc.VectorSubcoreMesh(core_axis_name="c", subcore_axis_name="s")

    @functools.partial(
        pl.kernel,
        out_type=jax.ShapeDtypeStruct((NC * NPAD, D), jnp.float32),
        mesh=mesh,
        scratch_types=[
            pltpu.VMEM((4, CHUNK), jnp.int32),
            pltpu.VMEM((CHUNK, D), jnp.float32),
            pltpu.VMEM((CHUNK, D), jnp.float32),
            pltpu.VMEM_SHARED((NPAD, D), jnp.float32),
            pltpu.SemaphoreType.DMA,
        ],
    )
    def spmm(h_hbm, edge_hbm, zeros_hbm, out_hbm,
             idx_v, rows_v, rows2_v, acc_sh, sem):
        c = lax.axis_index("c")
        s = lax.axis_index("s")
        wid = s * NC + c

        pltpu.sync_copy(zeros_hbm, acc_sh.at[pl.ds(s * ROWS_PER_TILE, ROWS_PER_TILE)])
        plsc.subcore_barrier()

        def body(t, carry):
            k = t * NW + wid

            @pl.when(k < NPAIR)
            def _():
                pltpu.sync_copy(edge_hbm.at[k], idx_v)
                pltpu.async_copy(h_hbm.at[idx_v.at[0]], rows_v, sem)
                pltpu.async_copy(h_hbm.at[idx_v.at[2]], rows2_v, sem)
                pltpu.make_async_copy(h_hbm.at[idx_v.at[0]], rows_v, sem).wait()
                pltpu.sync_copy(rows_v, acc_sh.at[idx_v.at[1]], add=True)
                pltpu.make_async_copy(h_hbm.at[idx_v.at[2]], rows2_v, sem).wait()
                pltpu.sync_copy(rows2_v, acc_sh.at[idx_v.at[3]], add=True)

            return carry

        lax.fori_loop(0, TRIPS, body, 0)
        plsc.subcore_barrier()

        pltpu.sync_copy(
            acc_sh.at[pl.ds(s * ROWS_PER_TILE, ROWS_PER_TILE)],
            out_hbm.at[pl.ds(c * NPAD + s * ROWS_PER_TILE, ROWS_PER_TILE)],
        )

    return spmm(h, edges, zeros)


_R = 1000


def _mm_body(x_ref, w_ref, o_ref):
    o_ref[...] = jnp.dot(x_ref[...], w_ref[...],
                         preferred_element_type=jnp.float32, precision=_PREC)


def _mm(x, w):
    return pl.pallas_call(
        _mm_body,
        grid=(N // _R,),
        in_specs=[
            pl.BlockSpec((_R, D), lambda i: (i, 0)),
            pl.BlockSpec((D, D), lambda i: (0, 0)),
        ],
        out_specs=pl.BlockSpec((_R, D), lambda i: (i, 0)),
        out_shape=jax.ShapeDtypeStruct((N, D), jnp.float32),
    )(x, w)


def _combine_mm_body(p0_ref, p1_ref, b_ref, w_ref, o_ref):
    h = jax.nn.relu(p0_ref[...] + p1_ref[...] + b_ref[...])
    o_ref[...] = jnp.dot(h, w_ref[...],
                         preferred_element_type=jnp.float32, precision=_PREC)


def _combine_mm(p0, p1, b, w):
    return pl.pallas_call(
        _combine_mm_body,
        grid=(N // _R,),
        in_specs=[
            pl.BlockSpec((_R, D), lambda i: (i, 0)),
            pl.BlockSpec((_R, D), lambda i: (i, 0)),
            pl.BlockSpec((1, D), lambda i: (0, 0)),
            pl.BlockSpec((D, D), lambda i: (0, 0)),
        ],
        out_specs=pl.BlockSpec((_R, D), lambda i: (i, 0)),
        out_shape=jax.ShapeDtypeStruct((N, D), jnp.float32),
    )(p0, p1, b, w)


def _ff_body(q0_ref, q1_ref, b2_ref, wf1_ref, bf1_ref, wf2_ref, bf2_ref,
             wf3_ref, bf3_ref, ws_ref, bs_ref, o_ref):
    h = q0_ref[...] + q1_ref[...] + b2_ref[...]
    blk = jax.nn.relu(jnp.dot(h, wf1_ref[...],
                              preferred_element_type=jnp.float32,
                              precision=_PREC) + bf1_ref[...])
    blk = jax.nn.relu(jnp.dot(blk, wf2_ref[...],
                              preferred_element_type=jnp.float32,
                              precision=_PREC) + bf2_ref[...])
    blk = jnp.dot(blk, wf3_ref[...],
                  preferred_element_type=jnp.float32, precision=_PREC) + bf3_ref[...]
    sc = jnp.dot(h, ws_ref[...],
                 preferred_element_type=jnp.float32, precision=_PREC) + bs_ref[...]
    o_ref[...] = blk + sc


def _ff(q0, q1, b2, wf1, bf1, wf2, bf2, wf3, bf3, ws, bs):
    row = pl.BlockSpec((_R, D), lambda i: (i, 0))
    mat = pl.BlockSpec((D, D), lambda i: (0, 0))
    vec = pl.BlockSpec((1, D), lambda i: (0, 0))
    return pl.pallas_call(
        _ff_body,
        grid=(N // _R,),
        in_specs=[row, row, vec, mat, vec, mat, vec, mat, vec, mat, vec],
        out_specs=row,
        out_shape=jax.ShapeDtypeStruct((N, D), jnp.float32),
    )(q0, q1, b2, wf1, bf1, wf2, bf2, wf3, bf3, ws, bs)


def kernel(x, edge_index, W1, b1, W2, b2, Wf1, bf1, Wf2, bf2, Wf3, bf3, Ws, bs):
    src2 = edge_index[0].reshape(NPAIR, 2, CHUNK)
    dst2 = edge_index[1].reshape(NPAIR, 2, CHUNK)
    edges = jnp.stack(
        [src2[:, 0], dst2[:, 0], src2[:, 1], dst2[:, 1]], axis=1)
    zeros = jnp.zeros((ROWS_PER_TILE, D), jnp.float32)

    h1 = _mm(x, W1)
    p = _spmm_sc(h1, edges, zeros)
    h2 = _combine_mm(p[:N], p[NPAD:NPAD + N], b1.reshape(1, D), W2)
    q = _spmm_sc(h2, edges, zeros)
    out = _ff(q[:N], q[NPAD:NPAD + N], b2.reshape(1, D),
              Wf1, bf1.reshape(1, D), Wf2, bf2.reshape(1, D),
              Wf3, bf3.reshape(1, D), Ws, bs.reshape(1, D))
    return out


# speedup vs baseline: 3.0057x; 1.0954x over previous
import functools

import jax
import jax.numpy as jnp
from jax import lax
from jax.experimental import pallas as pl
from jax.experimental.pallas import tpu as pltpu
from jax.experimental.pallas import tpu_sc as plsc

N = 10000
E = 320000
D = 128

NC = 2
NS = 16
NW = NC * NS

CHUNK = 128
NQUAD = E // (4 * CHUNK)
TRIPS = -(-NQUAD // NW)
NPAD = 10240
ROWS_PER_TILE = NPAD // NS

_PREC = jax.lax.Precision.HIGHEST


def _spmm_sc(h, edges, zeros):
    mesh = plsc.VectorSubcoreMesh(core_axis_name="c", subcore_axis_name="s")

    @functools.partial(
        pl.kernel,
        out_type=jax.ShapeDtypeStruct((NC * NPAD, D), jnp.float32),
        mesh=mesh,
        scratch_types=[
            pltpu.VMEM((8, CHUNK), jnp.int32),
            pltpu.VMEM((CHUNK, D), jnp.float32),
            pltpu.VMEM((CHUNK, D), jnp.float32),
            pltpu.VMEM_SHARED((NPAD, D), jnp.float32),
            pltpu.SemaphoreType.DMA,
        ],
    )
    def spmm(h_hbm, edge_hbm, zeros_hbm, out_hbm,
             idx_v, rows_v, rows2_v, acc_sh, sem):
        c = lax.axis_index("c")
        s = lax.axis_index("s")
        wid = s * NC + c

        pltpu.sync_copy(zeros_hbm, acc_sh.at[pl.ds(s * ROWS_PER_TILE, ROWS_PER_TILE)])
        plsc.subcore_barrier()

        def body(t, carry):
            k = t * NW + wid

            @pl.when(k < NQUAD)
            def _():
                pltpu.sync_copy(edge_hbm.at[k], idx_v)
                pltpu.async_copy(h_hbm.at[idx_v.at[0]], rows_v, sem)
                pltpu.async_copy(h_hbm.at[idx_v.at[2]], rows2_v, sem)
                pltpu.make_async_copy(h_hbm.at[idx_v.at[0]], rows_v, sem).wait()
                pltpu.sync_copy(rows_v, acc_sh.at[idx_v.at[1]], add=True)
                pltpu.async_copy(h_hbm.at[idx_v.at[4]], rows_v, sem)
                pltpu.make_async_copy(h_hbm.at[idx_v.at[2]], rows2_v, sem).wait()
                pltpu.sync_copy(rows2_v, acc_sh.at[idx_v.at[3]], add=True)
                pltpu.async_copy(h_hbm.at[idx_v.at[6]], rows2_v, sem)
                pltpu.make_async_copy(h_hbm.at[idx_v.at[4]], rows_v, sem).wait()
                pltpu.sync_copy(rows_v, acc_sh.at[idx_v.at[5]], add=True)
                pltpu.make_async_copy(h_hbm.at[idx_v.at[6]], rows2_v, sem).wait()
                pltpu.sync_copy(rows2_v, acc_sh.at[idx_v.at[7]], add=True)

            return carry

        lax.fori_loop(0, TRIPS, body, 0)
        plsc.subcore_barrier()

        pltpu.sync_copy(
            acc_sh.at[pl.ds(s * ROWS_PER_TILE, ROWS_PER_TILE)],
            out_hbm.at[pl.ds(c * NPAD + s * ROWS_PER_TILE, ROWS_PER_TILE)],
        )

    return spmm(h, edges, zeros)


_R = 1000


def _mm_body(x_ref, w_ref, o_ref):
    o_ref[...] = jnp.dot(x_ref[...], w_ref[...],
                         preferred_element_type=jnp.float32, precision=_PREC)


def _mm(x, w):
    return pl.pallas_call(
        _mm_body,
        grid=(N // _R,),
        in_specs=[
            pl.BlockSpec((_R, D), lambda i: (i, 0)),
            pl.BlockSpec((D, D), lambda i: (0, 0)),
        ],
        out_specs=pl.BlockSpec((_R, D), lambda i: (i, 0)),
        out_shape=jax.ShapeDtypeStruct((N, D), jnp.float32),
    )(x, w)


def _combine_mm_body(p0_ref, p1_ref, b_ref, w_ref, o_ref):
    h = jax.nn.relu(p0_ref[...] + p1_ref[...] + b_ref[...])
    o_ref[...] = jnp.dot(h, w_ref[...],
                         preferred_element_type=jnp.float32, precision=_PREC)


def _combine_mm(p0, p1, b, w):
    return pl.pallas_call(
        _combine_mm_body,
        grid=(N // _R,),
        in_specs=[
            pl.BlockSpec((_R, D), lambda i: (i, 0)),
            pl.BlockSpec((_R, D), lambda i: (i, 0)),
            pl.BlockSpec((1, D), lambda i: (0, 0)),
            pl.BlockSpec((D, D), lambda i: (0, 0)),
        ],
        out_specs=pl.BlockSpec((_R, D), lambda i: (i, 0)),
        out_shape=jax.ShapeDtypeStruct((N, D), jnp.float32),
    )(p0, p1, b, w)


def _ff_body(q0_ref, q1_ref, b2_ref, wf1_ref, bf1_ref, wf2_ref, bf2_ref,
             wf3_ref, bf3_ref, ws_ref, bs_ref, o_ref):
    h = q0_ref[...] + q1_ref[...] + b2_ref[...]
    blk = jax.nn.relu(jnp.dot(h, wf1_ref[...],
                              preferred_element_type=jnp.float32,
                              precision=_PREC) + bf1_ref[...])
    blk = jax.nn.relu(jnp.dot(blk, wf2_ref[...],
                              preferred_element_type=jnp.float32,
                              precision=_PREC) + bf2_ref[...])
    blk = jnp.dot(blk, wf3_ref[...],
                  preferred_element_type=jnp.float32, precision=_PREC) + bf3_ref[...]
    sc = jnp.dot(h, ws_ref[...],
                 preferred_element_type=jnp.float32, precision=_PREC) + bs_ref[...]
    o_ref[...] = blk + sc


def _ff(q0, q1, b2, wf1, bf1, wf2, bf2, wf3, bf3, ws, bs):
    row = pl.BlockSpec((_R, D), lambda i: (i, 0))
    mat = pl.BlockSpec((D, D), lambda i: (0, 0))
    vec = pl.BlockSpec((1, D), lambda i: (0, 0))
    return pl.pallas_call(
        _ff_body,
        grid=(N // _R,),
        in_specs=[row, row, vec, mat, vec, mat, vec, mat, vec, mat, vec],
        out_specs=row,
        out_shape=jax.ShapeDtypeStruct((N, D), jnp.float32),
    )(q0, q1, b2, wf1, bf1, wf2, bf2, wf3, bf3, ws, bs)


def kernel(x, edge_index, W1, b1, W2, b2, Wf1, bf1, Wf2, bf2, Wf3, bf3, Ws, bs):
    src2 = edge_index[0].reshape(NQUAD, 4, 1, CHUNK)
    dst2 = edge_index[1].reshape(NQUAD, 4, 1, CHUNK)
    edges = jnp.concatenate([src2, dst2], axis=2).reshape(NQUAD, 8, CHUNK)
    zeros = jnp.zeros((ROWS_PER_TILE, D), jnp.float32)

    h1 = _mm(x, W1)
    p = _spmm_sc(h1, edges, zeros)
    h2 = _combine_mm(p[:N], p[NPAD:NPAD + N], b1.reshape(1, D), W2)
    q = _spmm_sc(h2, edges, zeros)
    out = _ff(q[:N], q[NPAD:NPAD + N], b2.reshape(1, D),
              Wf1, bf1.reshape(1, D), Wf2, bf2.reshape(1, D),
              Wf3, bf3.reshape(1, D), Ws, bs.reshape(1, D))
    return out
